# Initial kernel scaffold; baseline (speedup 1.0000x reference)
import jax
import jax.numpy as jnp
from jax.experimental import pallas as pl


def kernel(x, edge_index, corp_x, corp_edge_index, nb_negative_index, negative_index, noise, W1, b1, W2, b2, Wstd, bstd, bi_weights):
    raise NotImplementedError("write your pallas kernel here")


# speedup vs baseline: 1.7818x; 1.7818x over previous
import functools

import jax
import jax.numpy as jnp
from jax import lax
from jax.experimental import pallas as pl
from jax.experimental.pallas import tpu as pltpu
from jax.experimental.pallas import tpu_sc as plsc

N = 10000
D = 128
E = 320000
NEG = 10
AUG = 2
AUG_DGI_W = 1e-05
INS_W = 1e-05
NORM_W = -0.1
HINGE_W = 0.0
L2R = 1e-05

_NC = 2
_NS = 16
_NW = _NC * _NS


def _segsum_sc(g, src, dst):
    epw = E // _NW
    C = 80
    iters = epw // C
    NP = 10240
    rpt = NP // _NS
    mesh = plsc.VectorSubcoreMesh(core_axis_name="c", subcore_axis_name="s")

    @functools.partial(
        pl.kernel,
        out_type=jax.ShapeDtypeStruct((_NC, NP, D), jnp.float32),
        mesh=mesh,
        scratch_types=[
            pltpu.VMEM((C,), jnp.int32),
            pltpu.VMEM((C,), jnp.int32),
            pltpu.VMEM((C, D), jnp.float32),
            pltpu.VMEM_SHARED((NP, D), jnp.float32),
            pltpu.SemaphoreType.DMA,
        ],
    )
    def k(g_hbm, src_hbm, dst_hbm, out_hbm, sidx, didx, rows, acc, sem):
        cid = lax.axis_index("c")
        sid = lax.axis_index("s")
        wid = sid * _NC + cid

        def zb(i, carry):
            r = i // (D // 16)
            l = i % (D // 16)
            rows[r, pl.ds(l * 16, 16)] = jnp.zeros((16,), jnp.float32)
            return carry
        lax.fori_loop(0, C * (D // 16), zb, 0)
        for j in range(rpt // C):
            pltpu.sync_copy(rows, acc.at[pl.ds(sid * rpt + j * C, C)])
        plsc.subcore_barrier()

        base = wid * epw

        def body(i, carry):
            off = base + i * C
            pltpu.sync_copy(src_hbm.at[pl.ds(off, C)], sidx)
            pltpu.sync_copy(dst_hbm.at[pl.ds(off, C)], didx)
            pltpu.async_copy(g_hbm.at[sidx], rows, sem).wait()
            pltpu.sync_copy(rows, acc.at[didx], add=True)
            return carry
        lax.fori_loop(0, iters, body, 0)
        plsc.subcore_barrier()

        pltpu.sync_copy(acc.at[pl.ds(sid * rpt, rpt)],
                        out_hbm.at[cid, pl.ds(sid * rpt, rpt)])

    return k(g, src, dst)


def _gather_sc(h, idx):
    R = idx.shape[0]
    rpw = R // _NW
    C = 80
    iters = rpw // C
    mesh = plsc.VectorSubcoreMesh(core_axis_name="c", subcore_axis_name="s")

    @functools.partial(
        pl.kernel,
        out_type=jax.ShapeDtypeStruct((R, D), jnp.float32),
        mesh=mesh,
        scratch_types=[
            pltpu.VMEM((C,), jnp.int32),
            pltpu.VMEM((C, D), jnp.float32),
            pltpu.SemaphoreType.DMA,
        ],
    )
    def k(h_hbm, idx_hbm, out_hbm, vidx, rows, sem):
        cid = lax.axis_index("c")
        sid = lax.axis_index("s")
        wid = sid * _NC + cid
        base = wid * rpw

        def body(i, carry):
            off = base + i * C
            pltpu.sync_copy(idx_hbm.at[pl.ds(off, C)], vidx)
            pltpu.async_copy(h_hbm.at[vidx], rows, sem).wait()
            pltpu.sync_copy(rows, out_hbm.at[pl.ds(off, C)])
            return carry
        lax.fori_loop(0, iters, body, 0)

    return k(h, idx)


_BM = 1000


def _mm_kernel(x_ref, w_ref, o_ref):
    o_ref[...] = jnp.dot(x_ref[...], w_ref[...],
                         preferred_element_type=jnp.float32)


def _mm(x, W):
    M = x.shape[0]
    return pl.pallas_call(
        _mm_kernel,
        grid=(M // _BM,),
        in_specs=[pl.BlockSpec((_BM, D), lambda i: (i, 0)),
                  pl.BlockSpec((D, D), lambda i: (0, 0))],
        out_specs=pl.BlockSpec((_BM, D), lambda i: (i, 0)),
        out_shape=jax.ShapeDtypeStruct((M, D), jnp.float32),
    )(x, W)


def _relu_mm_kernel(p_ref, b_ref, w_ref, o_ref):
    a = jnp.maximum(p_ref[0] + p_ref[1] + b_ref[...], 0.0)
    o_ref[...] = jnp.dot(a, w_ref[...], preferred_element_type=jnp.float32)


def _relu_mm(p, b_row, W):
    return pl.pallas_call(
        _relu_mm_kernel,
        grid=(N // _BM,),
        in_specs=[pl.BlockSpec((2, _BM, D), lambda i: (0, i, 0)),
                  pl.BlockSpec((1, D), lambda i: (0, 0)),
                  pl.BlockSpec((D, D), lambda i: (0, 0))],
        out_specs=pl.BlockSpec((_BM, D), lambda i: (i, 0)),
        out_shape=jax.ShapeDtypeStruct((N, D), jnp.float32),
    )(p, b_row, W)


def _finalize_h_kernel(q_ref, qc_ref, b_ref, h_ref, ch_ref, hs_ref):
    h = jnp.maximum(q_ref[0] + q_ref[1] + b_ref[...], 0.0)
    ch = jnp.maximum(qc_ref[0] + qc_ref[1] + b_ref[...], 0.0)
    h_ref[...] = h
    ch_ref[...] = ch

    @pl.when(pl.program_id(0) == 0)
    def _():
        hs_ref[...] = jnp.zeros_like(hs_ref)
    hs_ref[...] += jnp.sum(h, axis=0, keepdims=True)


def _finalize_h(q, qc, b_row):
    return pl.pallas_call(
        _finalize_h_kernel,
        grid=(N // _BM,),
        in_specs=[pl.BlockSpec((2, _BM, D), lambda i: (0, i, 0)),
                  pl.BlockSpec((2, _BM, D), lambda i: (0, i, 0)),
                  pl.BlockSpec((1, D), lambda i: (0, 0))],
        out_specs=[pl.BlockSpec((_BM, D), lambda i: (i, 0)),
                   pl.BlockSpec((_BM, D), lambda i: (i, 0)),
                   pl.BlockSpec((1, D), lambda i: (0, 0))],
        out_shape=[jax.ShapeDtypeStruct((N, D), jnp.float32),
                   jax.ShapeDtypeStruct((N, D), jnp.float32),
                   jax.ShapeDtypeStruct((1, D), jnp.float32)],
    )(q, qc, b_row)


def _ctx_kernel(hs_ref, bi_ref, wstd_ref, bstd_ref, v_ref, reg_ref):
    c = jax.nn.sigmoid(hs_ref[...] / float(N))
    v_ref[...] = jnp.sum(bi_ref[...] * c, axis=1, keepdims=True)
    reg_ref[...] = L2R * (jnp.sum(wstd_ref[...] ** 2)
                          + jnp.sum(bstd_ref[...] ** 2)).reshape(1, 1)


def _ctx(hsum, bi, wstd, bstd11):
    return pl.pallas_call(
        _ctx_kernel,
        out_shape=[jax.ShapeDtypeStruct((D, 1), jnp.float32),
                   jax.ShapeDtypeStruct((1, 1), jnp.float32)],
    )(hsum, bi, wstd, bstd11)


def _bce1(z):
    return jnp.maximum(z, 0.0) - z + jnp.log1p(jnp.exp(-jnp.abs(z)))


def _bce0(z):
    return jnp.maximum(z, 0.0) + jnp.log1p(jnp.exp(-jnp.abs(z)))


_BL = 400


def _loss_kernel(h_ref, ch_ref, n0_ref, n1_ref, ng0_ref, ng1_ref, nb_ref,
                 v_ref, wstd_ref, bstd_ref,
                 avg_ref, spos_ref, scorp_ref, sa0_ref, sa1_ref,
                 sins_ref, snorm_ref, shinge_ref):
    h = h_ref[...]
    ch = ch_ref[...]
    v = v_ref[...]
    std = jnp.maximum(jnp.dot(h, wstd_ref[...],
                              preferred_element_type=jnp.float32)
                      + bstd_ref[...], 0.0)

    s_pos = jnp.sum(_bce1(jnp.dot(h, v, preferred_element_type=jnp.float32)))
    s_corp = jnp.sum(_bce0(jnp.dot(ch, v, preferred_element_type=jnp.float32)))

    aug_sum = jnp.zeros_like(h)
    s_aug = [None, None]
    s_ins = 0.0
    s_norm = 0.0
    s_hinge = 0.0
    for i, (n_ref, ng_ref) in enumerate(((n0_ref, ng0_ref), (n1_ref, ng1_ref))):
        ns = n_ref[...]
        ns = ns / (jnp.sqrt(jnp.sum(ns * ns, axis=-1, keepdims=True)) + 1e-12)
        aug = h + ns * std
        aug_sum = aug_sum + aug
        s_aug[i] = jnp.sum(_bce1(jnp.dot(aug, v,
                                         preferred_element_type=jnp.float32)))
        nb = nb_ref[...]
        diff = aug[:, None, :] - nb
        d2 = jnp.sum(diff * diff, axis=-1)
        s_norm = s_norm + jnp.sum(jnp.maximum(d2 - std, 0.0))
        pos = jnp.sum(aug * h, axis=1, keepdims=True)
        negl = jnp.sum(aug[:, None, :] * ng_ref[...], axis=-1)
        s_ins = s_ins + jnp.sum(_bce1(pos)) + jnp.sum(_bce0(negl))
        s_hinge = s_hinge + jnp.sum(jnp.maximum(0.0, negl - pos))

    avg_ref[...] = aug_sum / float(AUG)

    @pl.when(pl.program_id(0) == 0)
    def _():
        for r in (spos_ref, scorp_ref, sa0_ref, sa1_ref,
                  sins_ref, snorm_ref, shinge_ref):
            r[...] = jnp.zeros_like(r)
    spos_ref[...] += s_pos.reshape(1, 1)
    scorp_ref[...] += s_corp.reshape(1, 1)
    sa0_ref[...] += s_aug[0].reshape(1, 1)
    sa1_ref[...] += s_aug[1].reshape(1, 1)
    sins_ref[...] += s_ins.reshape(1, 1)
    snorm_ref[...] += s_norm.reshape(1, 1)
    shinge_ref[...] += s_hinge.reshape(1, 1)


def _loss(h, ch, n0, n1, ng0, ng1, nb, v, wstd, bstd11):
    row = lambda i: (i, 0)
    row3 = lambda i: (i, 0, 0)
    fix = lambda i: (0, 0)
    outs = pl.pallas_call(
        _loss_kernel,
        grid=(N // _BL,),
        in_specs=[pl.BlockSpec((_BL, D), row),
                  pl.BlockSpec((_BL, D), row),
                  pl.BlockSpec((_BL, D), row),
                  pl.BlockSpec((_BL, D), row),
                  pl.BlockSpec((_BL, NEG, D), row3),
                  pl.BlockSpec((_BL, NEG, D), row3),
                  pl.BlockSpec((_BL, NEG, D), row3),
                  pl.BlockSpec((D, 1), fix),
                  pl.BlockSpec((D, 1), fix),
                  pl.BlockSpec((1, 1), fix)],
        out_specs=[pl.BlockSpec((_BL, D), row)] + [pl.BlockSpec((1, 1), fix)] * 7,
        out_shape=[jax.ShapeDtypeStruct((N, D), jnp.float32)]
                  + [jax.ShapeDtypeStruct((1, 1), jnp.float32)] * 7,
    )(h, ch, n0, n1, ng0, ng1, nb, v, wstd, bstd11)
    return outs


def kernel(x, edge_index, corp_x, corp_edge_index, nb_negative_index,
           negative_index, noise, W1, b1, W2, b2, Wstd, bstd, bi_weights):
    edge_index = edge_index.astype(jnp.int32)
    corp_edge_index = corp_edge_index.astype(jnp.int32)
    e_src, e_dst = edge_index[0], edge_index[1]
    ce_src, ce_dst = corp_edge_index[0], corp_edge_index[1]
    b1_row = b1.reshape(1, D)
    b2_row = b2.reshape(1, D)
    bstd11 = bstd.reshape(1, 1)

    g1 = _mm(x, W1)
    g1c = _mm(corp_x, W1)
    p = _segsum_sc(g1, e_src, e_dst)
    pc = _segsum_sc(g1c, ce_src, ce_dst)
    g2 = _relu_mm(p, b1_row, W2)
    g2c = _relu_mm(pc, b1_row, W2)
    q = _segsum_sc(g2, e_src, e_dst)
    qc = _segsum_sc(g2c, ce_src, ce_dst)
    h, ch, hsum = _finalize_h(q, qc, b2_row)

    v, reg = _ctx(hsum, bi_weights, Wstd, bstd11)

    R = AUG * N * NEG + N * NEG
    RP = 307200
    idx = jnp.concatenate([
        negative_index.astype(jnp.int32).reshape(-1),
        nb_negative_index.astype(jnp.int32).reshape(-1),
        jnp.zeros((RP - R,), jnp.int32),
    ])
    rows = _gather_sc(h, idx)
    ng0 = rows[:N * NEG].reshape(N, NEG, D)
    ng1 = rows[N * NEG:2 * N * NEG].reshape(N, NEG, D)
    nb = rows[2 * N * NEG:3 * N * NEG].reshape(N, NEG, D)

    (aug_avg, s_pos, s_corp, s_a0, s_a1, s_ins, s_norm, s_hinge) = _loss(
        h, ch, noise[0], noise[1], ng0, ng1, nb, v, Wstd, bstd11)

    s_pos = s_pos[0, 0]
    s_corp = s_corp[0, 0]
    dgi = (s_pos + s_corp) / (2.0 * N)
    aug_dgi = ((s_a0[0, 0] + s_corp) / (2.0 * N)
               + (s_a1[0, 0] + s_corp) / (2.0 * N)) * AUG_DGI_W
    ins = s_ins[0, 0] / N * INS_W
    norml = s_norm[0, 0] / N * NORM_W
    hinge = s_hinge[0, 0] / N * HINGE_W
    loss = dgi + aug_dgi + ins + hinge + norml
    total = loss + reg[0, 0]
    return (total, dgi, aug_dgi, ins, hinge, norml, h, aug_avg)


# speedup vs baseline: 2.6671x; 1.4968x over previous
import functools

import jax
import jax.numpy as jnp
from jax import lax
from jax.experimental import pallas as pl
from jax.experimental.pallas import tpu as pltpu
from jax.experimental.pallas import tpu_sc as plsc

N = 10000
D = 128
E = 320000
NEG = 10
AUG = 2
AUG_DGI_W = 1e-05
INS_W = 1e-05
NORM_W = -0.1
HINGE_W = 0.0
L2R = 1e-05

_NC = 2
_NS = 16
_NW = _NC * _NS


def _segsum_sc(g, src, dst):
    epw = E // _NW
    C = 40
    NB = 5
    groups = epw // (C * NB)
    NP = 10240
    rpt = NP // _NS
    mesh = plsc.VectorSubcoreMesh(core_axis_name="c", subcore_axis_name="s")

    @functools.partial(
        pl.kernel,
        out_type=jax.ShapeDtypeStruct((_NC, NP, D), jnp.float32),
        mesh=mesh,
        scratch_types=[
            pltpu.VMEM((NB, 2, C), jnp.int32),
            pltpu.VMEM((NB, 2, C), jnp.int32),
            pltpu.VMEM((NB, C, D), jnp.float32),
            pltpu.VMEM_SHARED((NP, D), jnp.float32),
            pltpu.SemaphoreType.DMA((NB,)),
            pltpu.SemaphoreType.DMA((NB,)),
            pltpu.SemaphoreType.DMA((NB,)),
        ],
    )
    def k(g_hbm, src_hbm, dst_hbm, out_hbm, sidx, didx, rows, acc,
          semi, semg, sems):
        cid = lax.axis_index("c")
        sid = lax.axis_index("s")
        wid = sid * _NC + cid

        def zb(i, carry):
            r = i // (D // 16)
            l = i % (D // 16)
            rows[0, r, pl.ds(l * 16, 16)] = jnp.zeros((16,), jnp.float32)
            return carry
        lax.fori_loop(0, C * (D // 16), zb, 0)
        for j in range(rpt // C):
            pltpu.sync_copy(rows.at[0], acc.at[pl.ds(sid * rpt + j * C, C)])
        plsc.subcore_barrier()

        base = wid * epw

        for b in range(NB):
            off = base + b * C
            pltpu.async_copy(src_hbm.at[pl.ds(off, C)], sidx.at[b, 0],
                             semi.at[b])
            pltpu.async_copy(dst_hbm.at[pl.ds(off, C)], didx.at[b, 0],
                             semi.at[b])

        def group(g, carry):
            a = lax.rem(g, 2)
            for b in range(NB):
                off = base + (g * NB + b) * C
                pltpu.make_async_copy(
                    src_hbm.at[pl.ds(off, C)], sidx.at[b, a],
                    semi.at[b]).wait()
                pltpu.make_async_copy(
                    dst_hbm.at[pl.ds(off, C)], didx.at[b, a],
                    semi.at[b]).wait()
            gd = [pltpu.async_copy(g_hbm.at[sidx.at[b, a]], rows.at[b],
                                   semg.at[b]) for b in range(NB)]
            @pl.when(g + 1 < groups)
            def _():
                for b in range(NB):
                    noff = base + ((g + 1) * NB + b) * C
                    pltpu.async_copy(src_hbm.at[pl.ds(noff, C)],
                                     sidx.at[b, 1 - a], semi.at[b])
                    pltpu.async_copy(dst_hbm.at[pl.ds(noff, C)],
                                     didx.at[b, 1 - a], semi.at[b])
            sd = []
            for b in range(NB):
                gd[b].wait()
                sd.append(pltpu.async_copy(rows.at[b], acc.at[didx.at[b, a]],
                                           sems.at[b], add=True))
            for d in sd:
                d.wait()
            return carry
        lax.fori_loop(0, groups, group, 0)
        plsc.subcore_barrier()

        pltpu.sync_copy(acc.at[pl.ds(sid * rpt, rpt)],
                        out_hbm.at[cid, pl.ds(sid * rpt, rpt)])

    return k(g, src, dst)


def _gather_sc(h, idx):
    R = idx.shape[0]
    rpw = R // _NW
    C = 128
    NB = 5
    groups = rpw // (C * NB)
    mesh = plsc.VectorSubcoreMesh(core_axis_name="c", subcore_axis_name="s")

    @functools.partial(
        pl.kernel,
        out_type=jax.ShapeDtypeStruct((R, D), jnp.float32),
        mesh=mesh,
        scratch_types=[
            pltpu.VMEM((NB, 2, C), jnp.int32),
            pltpu.VMEM((NB, C, D), jnp.float32),
            pltpu.SemaphoreType.DMA((NB,)),
            pltpu.SemaphoreType.DMA((NB,)),
            pltpu.SemaphoreType.DMA((NB,)),
        ],
    )
    def k(h_hbm, idx_hbm, out_hbm, vidx, rows, semi, semg, semw):
        cid = lax.axis_index("c")
        sid = lax.axis_index("s")
        wid = sid * _NC + cid
        base = wid * rpw

        for b in range(NB):
            pltpu.async_copy(idx_hbm.at[pl.ds(base + b * C, C)],
                             vidx.at[b, 0], semi.at[b])

        def group(g, carry):
            a = lax.rem(g, 2)
            for b in range(NB):
                off = base + (g * NB + b) * C
                pltpu.make_async_copy(
                    idx_hbm.at[pl.ds(off, C)], vidx.at[b, a],
                    semi.at[b]).wait()
            gd = [pltpu.async_copy(h_hbm.at[vidx.at[b, a]], rows.at[b],
                                   semg.at[b]) for b in range(NB)]
            @pl.when(g + 1 < groups)
            def _():
                for b in range(NB):
                    noff = base + ((g + 1) * NB + b) * C
                    pltpu.async_copy(idx_hbm.at[pl.ds(noff, C)],
                                     vidx.at[b, 1 - a], semi.at[b])
            wd = []
            for b in range(NB):
                gd[b].wait()
                off = base + (g * NB + b) * C
                wd.append(pltpu.async_copy(rows.at[b],
                                           out_hbm.at[pl.ds(off, C)],
                                           semw.at[b]))
            for d in wd:
                d.wait()
            return carry
        lax.fori_loop(0, groups, group, 0)

    return k(h, idx)


_BM = 1000


def _mm_kernel(x_ref, w_ref, o_ref):
    o_ref[...] = jnp.dot(x_ref[...], w_ref[...],
                         preferred_element_type=jnp.float32)


def _mm(x, W):
    M = x.shape[0]
    return pl.pallas_call(
        _mm_kernel,
        grid=(M // _BM,),
        in_specs=[pl.BlockSpec((_BM, D), lambda i: (i, 0)),
                  pl.BlockSpec((D, D), lambda i: (0, 0))],
        out_specs=pl.BlockSpec((_BM, D), lambda i: (i, 0)),
        out_shape=jax.ShapeDtypeStruct((M, D), jnp.float32),
    )(x, W)


def _relu_mm_kernel(p_ref, b_ref, w_ref, o_ref):
    a = jnp.maximum(p_ref[0] + p_ref[1] + b_ref[...], 0.0)
    o_ref[...] = jnp.dot(a, w_ref[...], preferred_element_type=jnp.float32)


def _relu_mm(p, b_row, W):
    return pl.pallas_call(
        _relu_mm_kernel,
        grid=(N // _BM,),
        in_specs=[pl.BlockSpec((2, _BM, D), lambda i: (0, i, 0)),
                  pl.BlockSpec((1, D), lambda i: (0, 0)),
                  pl.BlockSpec((D, D), lambda i: (0, 0))],
        out_specs=pl.BlockSpec((_BM, D), lambda i: (i, 0)),
        out_shape=jax.ShapeDtypeStruct((N, D), jnp.float32),
    )(p, b_row, W)


def _finalize_h_kernel(q_ref, qc_ref, b_ref, h_ref, ch_ref, hs_ref):
    h = jnp.maximum(q_ref[0] + q_ref[1] + b_ref[...], 0.0)
    ch = jnp.maximum(qc_ref[0] + qc_ref[1] + b_ref[...], 0.0)
    h_ref[...] = h
    ch_ref[...] = ch

    @pl.when(pl.program_id(0) == 0)
    def _():
        hs_ref[...] = jnp.zeros_like(hs_ref)
    hs_ref[...] += jnp.sum(h, axis=0, keepdims=True)


def _finalize_h(q, qc, b_row):
    return pl.pallas_call(
        _finalize_h_kernel,
        grid=(N // _BM,),
        in_specs=[pl.BlockSpec((2, _BM, D), lambda i: (0, i, 0)),
                  pl.BlockSpec((2, _BM, D), lambda i: (0, i, 0)),
                  pl.BlockSpec((1, D), lambda i: (0, 0))],
        out_specs=[pl.BlockSpec((_BM, D), lambda i: (i, 0)),
                   pl.BlockSpec((_BM, D), lambda i: (i, 0)),
                   pl.BlockSpec((1, D), lambda i: (0, 0))],
        out_shape=[jax.ShapeDtypeStruct((N, D), jnp.float32),
                   jax.ShapeDtypeStruct((N, D), jnp.float32),
                   jax.ShapeDtypeStruct((1, D), jnp.float32)],
    )(q, qc, b_row)


def _ctx_kernel(hs_ref, wstd_ref, bstd_ref, c_ref, reg_ref):
    c_ref[...] = jax.nn.sigmoid(hs_ref[...] / float(N))
    reg_ref[...] = L2R * (jnp.sum(wstd_ref[...] ** 2)
                          + jnp.sum(bstd_ref[...] ** 2)).reshape(1, 1)


def _ctx(hsum, wstd, bstd11):
    return pl.pallas_call(
        _ctx_kernel,
        out_shape=[jax.ShapeDtypeStruct((1, D), jnp.float32),
                   jax.ShapeDtypeStruct((1, 1), jnp.float32)],
    )(hsum, wstd, bstd11)


def _bce1(z):
    return jnp.maximum(z, 0.0) - z + jnp.log1p(jnp.exp(-jnp.abs(z)))


def _bce0(z):
    return jnp.maximum(z, 0.0) + jnp.log1p(jnp.exp(-jnp.abs(z)))


_BL = 400


def _loss_kernel(h_ref, ch_ref, n0_ref, n1_ref, ng0_ref, ng1_ref, nb_ref,
                 c_ref, bi_ref, wstd_ref, bstd_ref,
                 avg_ref, spos_ref, scorp_ref, sa0_ref, sa1_ref,
                 sins_ref, snorm_ref, shinge_ref):
    h = h_ref[...]
    ch = ch_ref[...]
    c = c_ref[...]
    bi = bi_ref[...]
    std = jnp.maximum(jnp.dot(h, wstd_ref[...],
                              preferred_element_type=jnp.float32)
                      + bstd_ref[...], 0.0)

    s_pos = jnp.sum(_bce1(jnp.dot(jnp.dot(h, bi,
                                          preferred_element_type=jnp.float32),
                                  c, preferred_element_type=jnp.float32)))
    s_corp = jnp.sum(_bce0(jnp.dot(jnp.dot(ch, bi,
                                           preferred_element_type=jnp.float32),
                                   c, preferred_element_type=jnp.float32)))

    aug_sum = jnp.zeros_like(h)
    s_aug = [None, None]
    s_ins = 0.0
    s_norm = 0.0
    s_hinge = 0.0
    for i, (n_ref, ng_ref) in enumerate(((n0_ref, ng0_ref), (n1_ref, ng1_ref))):
        ns = n_ref[...]
        ns = ns / (jnp.sqrt(jnp.sum(ns * ns, axis=-1, keepdims=True)) + 1e-12)
        aug = h + ns * std
        aug_sum = aug_sum + aug
        s_aug[i] = jnp.sum(_bce1(jnp.dot(jnp.dot(aug, bi,
                                                 preferred_element_type=jnp.float32),
                                         c, preferred_element_type=jnp.float32)))
        nb = nb_ref[...]
        diff = aug[:, None, :] - nb
        d2 = jnp.sum(diff * diff, axis=-1)
        s_norm = s_norm + jnp.sum(jnp.maximum(d2 - std, 0.0))
        pos = jnp.sum(aug * h, axis=1, keepdims=True)
        negl = jnp.sum(aug[:, None, :] * ng_ref[...], axis=-1)
        s_ins = s_ins + jnp.sum(_bce1(pos)) + jnp.sum(_bce0(negl))
        s_hinge = s_hinge + jnp.sum(jnp.maximum(0.0, negl - pos))

    avg_ref[...] = aug_sum / float(AUG)

    @pl.when(pl.program_id(0) == 0)
    def _():
        for r in (spos_ref, scorp_ref, sa0_ref, sa1_ref,
                  sins_ref, snorm_ref, shinge_ref):
            r[...] = jnp.zeros_like(r)
    spos_ref[...] += s_pos.reshape(1, 1)
    scorp_ref[...] += s_corp.reshape(1, 1)
    sa0_ref[...] += s_aug[0].reshape(1, 1)
    sa1_ref[...] += s_aug[1].reshape(1, 1)
    sins_ref[...] += s_ins.reshape(1, 1)
    snorm_ref[...] += s_norm.reshape(1, 1)
    shinge_ref[...] += s_hinge.reshape(1, 1)


def _loss(h, ch, n0, n1, ng0, ng1, nb, c_col, bi, wstd, bstd11):
    row = lambda i: (i, 0)
    row3 = lambda i: (i, 0, 0)
    fix = lambda i: (0, 0)
    outs = pl.pallas_call(
        _loss_kernel,
        grid=(N // _BL,),
        in_specs=[pl.BlockSpec((_BL, D), row),
                  pl.BlockSpec((_BL, D), row),
                  pl.BlockSpec((_BL, D), row),
                  pl.BlockSpec((_BL, D), row),
                  pl.BlockSpec((_BL, NEG, D), row3),
                  pl.BlockSpec((_BL, NEG, D), row3),
                  pl.BlockSpec((_BL, NEG, D), row3),
                  pl.BlockSpec((D, 1), fix),
                  pl.BlockSpec((D, D), fix),
                  pl.BlockSpec((D, 1), fix),
                  pl.BlockSpec((1, 1), fix)],
        out_specs=[pl.BlockSpec((_BL, D), row)] + [pl.BlockSpec((1, 1), fix)] * 7,
        out_shape=[jax.ShapeDtypeStruct((N, D), jnp.float32)]
                  + [jax.ShapeDtypeStruct((1, 1), jnp.float32)] * 7,
    )(h, ch, n0, n1, ng0, ng1, nb, c_col, bi, wstd, bstd11)
    return outs


def kernel(x, edge_index, corp_x, corp_edge_index, nb_negative_index,
           negative_index, noise, W1, b1, W2, b2, Wstd, bstd, bi_weights):
    edge_index = edge_index.astype(jnp.int32)
    corp_edge_index = corp_edge_index.astype(jnp.int32)
    e_src, e_dst = edge_index[0], edge_index[1]
    ce_src, ce_dst = corp_edge_index[0], corp_edge_index[1]
    b1_row = b1.reshape(1, D)
    b2_row = b2.reshape(1, D)
    bstd11 = bstd.reshape(1, 1)

    g1 = _mm(x, W1)
    g1c = _mm(corp_x, W1)
    p = _segsum_sc(g1, e_src, e_dst)
    pc = _segsum_sc(g1c, ce_src, ce_dst)
    g2 = _relu_mm(p, b1_row, W2)
    g2c = _relu_mm(pc, b1_row, W2)
    q = _segsum_sc(g2, e_src, e_dst)
    qc = _segsum_sc(g2c, ce_src, ce_dst)
    h, ch, hsum = _finalize_h(q, qc, b2_row)

    c_row, reg = _ctx(hsum, Wstd, bstd11)
    c_col = c_row.reshape(D, 1)

    R = AUG * N * NEG + N * NEG
    RP = 307200
    idx = jnp.concatenate([
        negative_index.astype(jnp.int32).reshape(-1),
        nb_negative_index.astype(jnp.int32).reshape(-1),
        jnp.zeros((RP - R,), jnp.int32),
    ])
    rows = _gather_sc(h, idx)
    ng0 = rows[:N * NEG].reshape(N, NEG, D)
    ng1 = rows[N * NEG:2 * N * NEG].reshape(N, NEG, D)
    nb = rows[2 * N * NEG:3 * N * NEG].reshape(N, NEG, D)

    (aug_avg, s_pos, s_corp, s_a0, s_a1, s_ins, s_norm, s_hinge) = _loss(
        h, ch, noise[0], noise[1], ng0, ng1, nb, c_col, bi_weights, Wstd,
        bstd11)

    s_pos = s_pos[0, 0]
    s_corp = s_corp[0, 0]
    dgi = (s_pos + s_corp) / (2.0 * N)
    aug_dgi = ((s_a0[0, 0] + s_corp) / (2.0 * N)
               + (s_a1[0, 0] + s_corp) / (2.0 * N)) * AUG_DGI_W
    ins = s_ins[0, 0] / N * INS_W
    norml = s_norm[0, 0] / N * NORM_W
    hinge = s_hinge[0, 0] / N * HINGE_W
    loss = dgi + aug_dgi + ins + hinge + norml
    total = loss + reg[0, 0]
    return (total, dgi, aug_dgi, ins, hinge, norml, h, aug_avg)


# speedup vs baseline: 2.6677x; 1.0002x over previous
import functools

import jax
import jax.numpy as jnp
from jax import lax
from jax.experimental import pallas as pl
from jax.experimental.pallas import tpu as pltpu
from jax.experimental.pallas import tpu_sc as plsc

N = 10000
D = 128
E = 320000
NEG = 10
AUG = 2
AUG_DGI_W = 1e-05
INS_W = 1e-05
NORM_W = -0.1
HINGE_W = 0.0
L2R = 1e-05

_NC = 2
_NS = 16
_NW = _NC * _NS


def _segsum_sc(g, src, dst):
    epw = E // _NW
    C = 40
    NB = 5
    groups = epw // (C * NB)
    NP = 10240
    rpt = NP // _NS
    mesh = plsc.VectorSubcoreMesh(core_axis_name="c", subcore_axis_name="s")

    @functools.partial(
        pl.kernel,
        out_type=jax.ShapeDtypeStruct((_NC, NP, D), jnp.float32),
        mesh=mesh,
        scratch_types=[
            pltpu.VMEM((NB, 2, C), jnp.int32),
            pltpu.VMEM((NB, 2, C), jnp.int32),
            pltpu.VMEM((NB, C, D), jnp.float32),
            pltpu.VMEM_SHARED((NP, D), jnp.float32),
            pltpu.SemaphoreType.DMA((NB,)),
            pltpu.SemaphoreType.DMA((NB,)),
            pltpu.SemaphoreType.DMA((NB,)),
        ],
    )
    def k(g_hbm, src_hbm, dst_hbm, out_hbm, sidx, didx, rows, acc,
          semi, semg, sems):
        cid = lax.axis_index("c")
        sid = lax.axis_index("s")
        wid = sid * _NC + cid

        def zb(i, carry):
            r = i // (D // 16)
            l = i % (D // 16)
            rows[0, r, pl.ds(l * 16, 16)] = jnp.zeros((16,), jnp.float32)
            return carry
        lax.fori_loop(0, C * (D // 16), zb, 0)
        for j in range(rpt // C):
            pltpu.sync_copy(rows.at[0], acc.at[pl.ds(sid * rpt + j * C, C)])
        plsc.subcore_barrier()

        base = wid * epw

        for b in range(NB):
            off = base + b * C
            pltpu.async_copy(src_hbm.at[pl.ds(off, C)], sidx.at[b, 0],
                             semi.at[b])
            pltpu.async_copy(dst_hbm.at[pl.ds(off, C)], didx.at[b, 0],
                             semi.at[b])

        def group(g, carry):
            a = lax.rem(g, 2)
            for b in range(NB):
                off = base + (g * NB + b) * C
                pltpu.make_async_copy(
                    src_hbm.at[pl.ds(off, C)], sidx.at[b, a],
                    semi.at[b]).wait()
                pltpu.make_async_copy(
                    dst_hbm.at[pl.ds(off, C)], didx.at[b, a],
                    semi.at[b]).wait()
            gd = [pltpu.async_copy(g_hbm.at[sidx.at[b, a]], rows.at[b],
                                   semg.at[b]) for b in range(NB)]
            @pl.when(g + 1 < groups)
            def _():
                for b in range(NB):
                    noff = base + ((g + 1) * NB + b) * C
                    pltpu.async_copy(src_hbm.at[pl.ds(noff, C)],
                                     sidx.at[b, 1 - a], semi.at[b])
                    pltpu.async_copy(dst_hbm.at[pl.ds(noff, C)],
                                     didx.at[b, 1 - a], semi.at[b])
            sd = []
            for b in range(NB):
                gd[b].wait()
                sd.append(pltpu.async_copy(rows.at[b], acc.at[didx.at[b, a]],
                                           sems.at[b], add=True))
            for d in sd:
                d.wait()
            return carry
        lax.fori_loop(0, groups, group, 0)
        plsc.subcore_barrier()

        pltpu.sync_copy(acc.at[pl.ds(sid * rpt, rpt)],
                        out_hbm.at[cid, pl.ds(sid * rpt, rpt)])

    return k(g, src, dst)


def _gather_sc(h, idx):
    R = idx.shape[0]
    rpw = R // _NW
    C = 128
    NB = 5
    groups = rpw // (C * NB)
    mesh = plsc.VectorSubcoreMesh(core_axis_name="c", subcore_axis_name="s")

    @functools.partial(
        pl.kernel,
        out_type=jax.ShapeDtypeStruct((R, D), jnp.float32),
        mesh=mesh,
        scratch_types=[
            pltpu.VMEM((NB, 2, C), jnp.int32),
            pltpu.VMEM((NB, C, D), jnp.float32),
            pltpu.SemaphoreType.DMA((NB,)),
            pltpu.SemaphoreType.DMA((NB,)),
            pltpu.SemaphoreType.DMA((NB,)),
        ],
    )
    def k(h_hbm, idx_hbm, out_hbm, vidx, rows, semi, semg, semw):
        cid = lax.axis_index("c")
        sid = lax.axis_index("s")
        wid = sid * _NC + cid
        base = wid * rpw

        for b in range(NB):
            pltpu.async_copy(idx_hbm.at[pl.ds(base + b * C, C)],
                             vidx.at[b, 0], semi.at[b])

        def group(g, carry):
            a = lax.rem(g, 2)
            for b in range(NB):
                off = base + (g * NB + b) * C
                pltpu.make_async_copy(
                    idx_hbm.at[pl.ds(off, C)], vidx.at[b, a],
                    semi.at[b]).wait()
            @pl.when(g > 0)
            def _():
                for b in range(NB):
                    pltpu.make_async_copy(
                        rows.at[b], out_hbm.at[pl.ds(base, C)],
                        semw.at[b]).wait()
            gd = [pltpu.async_copy(h_hbm.at[vidx.at[b, a]], rows.at[b],
                                   semg.at[b]) for b in range(NB)]
            @pl.when(g + 1 < groups)
            def _():
                for b in range(NB):
                    noff = base + ((g + 1) * NB + b) * C
                    pltpu.async_copy(idx_hbm.at[pl.ds(noff, C)],
                                     vidx.at[b, 1 - a], semi.at[b])
            for b in range(NB):
                gd[b].wait()
                off = base + (g * NB + b) * C
                pltpu.async_copy(rows.at[b], out_hbm.at[pl.ds(off, C)],
                                 semw.at[b])
            return carry
        lax.fori_loop(0, groups, group, 0)
        for b in range(NB):
            pltpu.make_async_copy(rows.at[b], out_hbm.at[pl.ds(base, C)],
                                  semw.at[b]).wait()

    return k(h, idx)


_BM = 1000


def _mm_kernel(x_ref, w_ref, o_ref):
    o_ref[...] = jnp.dot(x_ref[...], w_ref[...],
                         preferred_element_type=jnp.float32)


def _mm(x, W):
    M = x.shape[0]
    return pl.pallas_call(
        _mm_kernel,
        grid=(M // _BM,),
        in_specs=[pl.BlockSpec((_BM, D), lambda i: (i, 0)),
                  pl.BlockSpec((D, D), lambda i: (0, 0))],
        out_specs=pl.BlockSpec((_BM, D), lambda i: (i, 0)),
        out_shape=jax.ShapeDtypeStruct((M, D), jnp.float32),
    )(x, W)


def _relu_mm_kernel(p_ref, b_ref, w_ref, o_ref):
    a = jnp.maximum(p_ref[0] + p_ref[1] + b_ref[...], 0.0)
    o_ref[...] = jnp.dot(a, w_ref[...], preferred_element_type=jnp.float32)


def _relu_mm(p, b_row, W):
    return pl.pallas_call(
        _relu_mm_kernel,
        grid=(N // _BM,),
        in_specs=[pl.BlockSpec((2, _BM, D), lambda i: (0, i, 0)),
                  pl.BlockSpec((1, D), lambda i: (0, 0)),
                  pl.BlockSpec((D, D), lambda i: (0, 0))],
        out_specs=pl.BlockSpec((_BM, D), lambda i: (i, 0)),
        out_shape=jax.ShapeDtypeStruct((N, D), jnp.float32),
    )(p, b_row, W)


def _finalize_h_kernel(q_ref, qc_ref, b_ref, h_ref, ch_ref, hs_ref):
    h = jnp.maximum(q_ref[0] + q_ref[1] + b_ref[...], 0.0)
    ch = jnp.maximum(qc_ref[0] + qc_ref[1] + b_ref[...], 0.0)
    h_ref[...] = h
    ch_ref[...] = ch

    @pl.when(pl.program_id(0) == 0)
    def _():
        hs_ref[...] = jnp.zeros_like(hs_ref)
    hs_ref[...] += jnp.sum(h, axis=0, keepdims=True)


def _finalize_h(q, qc, b_row):
    return pl.pallas_call(
        _finalize_h_kernel,
        grid=(N // _BM,),
        in_specs=[pl.BlockSpec((2, _BM, D), lambda i: (0, i, 0)),
                  pl.BlockSpec((2, _BM, D), lambda i: (0, i, 0)),
                  pl.BlockSpec((1, D), lambda i: (0, 0))],
        out_specs=[pl.BlockSpec((_BM, D), lambda i: (i, 0)),
                   pl.BlockSpec((_BM, D), lambda i: (i, 0)),
                   pl.BlockSpec((1, D), lambda i: (0, 0))],
        out_shape=[jax.ShapeDtypeStruct((N, D), jnp.float32),
                   jax.ShapeDtypeStruct((N, D), jnp.float32),
                   jax.ShapeDtypeStruct((1, D), jnp.float32)],
    )(q, qc, b_row)


def _ctx_kernel(hs_ref, wstd_ref, bstd_ref, c_ref, reg_ref):
    c_ref[...] = jax.nn.sigmoid(hs_ref[...] / float(N))
    reg_ref[...] = L2R * (jnp.sum(wstd_ref[...] ** 2)
                          + jnp.sum(bstd_ref[...] ** 2)).reshape(1, 1)


def _ctx(hsum, wstd, bstd11):
    return pl.pallas_call(
        _ctx_kernel,
        out_shape=[jax.ShapeDtypeStruct((1, D), jnp.float32),
                   jax.ShapeDtypeStruct((1, 1), jnp.float32)],
    )(hsum, wstd, bstd11)


def _bce1(z):
    return jnp.maximum(z, 0.0) - z + jnp.log1p(jnp.exp(-jnp.abs(z)))


def _bce0(z):
    return jnp.maximum(z, 0.0) + jnp.log1p(jnp.exp(-jnp.abs(z)))


_BL = 400


def _loss_kernel(h_ref, ch_ref, n0_ref, n1_ref, ng0_ref, ng1_ref, nb_ref,
                 c_ref, bi_ref, wstd_ref, bstd_ref,
                 avg_ref, spos_ref, scorp_ref, sa0_ref, sa1_ref,
                 sins_ref, snorm_ref, shinge_ref):
    h = h_ref[...]
    ch = ch_ref[...]
    c = c_ref[...]
    bi = bi_ref[...]
    std = jnp.maximum(jnp.dot(h, wstd_ref[...],
                              preferred_element_type=jnp.float32)
                      + bstd_ref[...], 0.0)

    s_pos = jnp.sum(_bce1(jnp.dot(jnp.dot(h, bi,
                                          preferred_element_type=jnp.float32),
                                  c, preferred_element_type=jnp.float32)))
    s_corp = jnp.sum(_bce0(jnp.dot(jnp.dot(ch, bi,
                                           preferred_element_type=jnp.float32),
                                   c, preferred_element_type=jnp.float32)))

    aug_sum = jnp.zeros_like(h)
    s_aug = [None, None]
    s_ins = 0.0
    s_norm = 0.0
    s_hinge = 0.0
    for i, (n_ref, ng_ref) in enumerate(((n0_ref, ng0_ref), (n1_ref, ng1_ref))):
        ns = n_ref[...]
        ns = ns / (jnp.sqrt(jnp.sum(ns * ns, axis=-1, keepdims=True)) + 1e-12)
        aug = h + ns * std
        aug_sum = aug_sum + aug
        s_aug[i] = jnp.sum(_bce1(jnp.dot(jnp.dot(aug, bi,
                                                 preferred_element_type=jnp.float32),
                                         c, preferred_element_type=jnp.float32)))
        nb = nb_ref[...]
        diff = aug[:, None, :] - nb
        d2 = jnp.sum(diff * diff, axis=-1)
        s_norm = s_norm + jnp.sum(jnp.maximum(d2 - std, 0.0))
        pos = jnp.sum(aug * h, axis=1, keepdims=True)
        negl = jnp.sum(aug[:, None, :] * ng_ref[...], axis=-1)
        s_ins = s_ins + jnp.sum(_bce1(pos)) + jnp.sum(_bce0(negl))
        s_hinge = s_hinge + jnp.sum(jnp.maximum(0.0, negl - pos))

    avg_ref[...] = aug_sum / float(AUG)

    @pl.when(pl.program_id(0) == 0)
    def _():
        for r in (spos_ref, scorp_ref, sa0_ref, sa1_ref,
                  sins_ref, snorm_ref, shinge_ref):
            r[...] = jnp.zeros_like(r)
    spos_ref[...] += s_pos.reshape(1, 1)
    scorp_ref[...] += s_corp.reshape(1, 1)
    sa0_ref[...] += s_aug[0].reshape(1, 1)
    sa1_ref[...] += s_aug[1].reshape(1, 1)
    sins_ref[...] += s_ins.reshape(1, 1)
    snorm_ref[...] += s_norm.reshape(1, 1)
    shinge_ref[...] += s_hinge.reshape(1, 1)


def _loss(h, ch, n0, n1, ng0, ng1, nb, c_col, bi, wstd, bstd11):
    row = lambda i: (i, 0)
    row3 = lambda i: (i, 0, 0)
    fix = lambda i: (0, 0)
    outs = pl.pallas_call(
        _loss_kernel,
        grid=(N // _BL,),
        in_specs=[pl.BlockSpec((_BL, D), row),
                  pl.BlockSpec((_BL, D), row),
                  pl.BlockSpec((_BL, D), row),
                  pl.BlockSpec((_BL, D), row),
                  pl.BlockSpec((_BL, NEG, D), row3),
                  pl.BlockSpec((_BL, NEG, D), row3),
                  pl.BlockSpec((_BL, NEG, D), row3),
                  pl.BlockSpec((D, 1), fix),
                  pl.BlockSpec((D, D), fix),
                  pl.BlockSpec((D, 1), fix),
                  pl.BlockSpec((1, 1), fix)],
        out_specs=[pl.BlockSpec((_BL, D), row)] + [pl.BlockSpec((1, 1), fix)] * 7,
        out_shape=[jax.ShapeDtypeStruct((N, D), jnp.float32)]
                  + [jax.ShapeDtypeStruct((1, 1), jnp.float32)] * 7,
    )(h, ch, n0, n1, ng0, ng1, nb, c_col, bi, wstd, bstd11)
    return outs


def kernel(x, edge_index, corp_x, corp_edge_index, nb_negative_index,
           negative_index, noise, W1, b1, W2, b2, Wstd, bstd, bi_weights):
    edge_index = edge_index.astype(jnp.int32)
    corp_edge_index = corp_edge_index.astype(jnp.int32)
    e_src, e_dst = edge_index[0], edge_index[1]
    ce_src, ce_dst = corp_edge_index[0], corp_edge_index[1]
    b1_row = b1.reshape(1, D)
    b2_row = b2.reshape(1, D)
    bstd11 = bstd.reshape(1, 1)

    g1 = _mm(x, W1)
    g1c = _mm(corp_x, W1)
    p = _segsum_sc(g1, e_src, e_dst)
    pc = _segsum_sc(g1c, ce_src, ce_dst)
    g2 = _relu_mm(p, b1_row, W2)
    g2c = _relu_mm(pc, b1_row, W2)
    q = _segsum_sc(g2, e_src, e_dst)
    qc = _segsum_sc(g2c, ce_src, ce_dst)
    h, ch, hsum = _finalize_h(q, qc, b2_row)

    c_row, reg = _ctx(hsum, Wstd, bstd11)
    c_col = c_row.reshape(D, 1)

    R = AUG * N * NEG + N * NEG
    RP = 307200
    idx = jnp.concatenate([
        negative_index.astype(jnp.int32).reshape(-1),
        nb_negative_index.astype(jnp.int32).reshape(-1),
        jnp.zeros((RP - R,), jnp.int32),
    ])
    rows = _gather_sc(h, idx)
    ng0 = rows[:N * NEG].reshape(N, NEG, D)
    ng1 = rows[N * NEG:2 * N * NEG].reshape(N, NEG, D)
    nb = rows[2 * N * NEG:3 * N * NEG].reshape(N, NEG, D)

    (aug_avg, s_pos, s_corp, s_a0, s_a1, s_ins, s_norm, s_hinge) = _loss(
        h, ch, noise[0], noise[1], ng0, ng1, nb, c_col, bi_weights, Wstd,
        bstd11)

    s_pos = s_pos[0, 0]
    s_corp = s_corp[0, 0]
    dgi = (s_pos + s_corp) / (2.0 * N)
    aug_dgi = ((s_a0[0, 0] + s_corp) / (2.0 * N)
               + (s_a1[0, 0] + s_corp) / (2.0 * N)) * AUG_DGI_W
    ins = s_ins[0, 0] / N * INS_W
    norml = s_norm[0, 0] / N * NORM_W
    hinge = s_hinge[0, 0] / N * HINGE_W
    loss = dgi + aug_dgi + ins + hinge + norml
    total = loss + reg[0, 0]
    return (total, dgi, aug_dgi, ins, hinge, norml, h, aug_avg)
